# baseline (speedup 1.0000x reference)
import functools

import jax
import jax.numpy as jnp
from jax import lax
from jax.experimental import pallas as pl
from jax.experimental.pallas import tpu as pltpu
from jax.experimental.pallas import tpu_sc as plsc

N = 10000
E = 320000
D = 128
NC = 2
NS = 16
NT = NC * NS
CH = 128
CPT = 80
EPT = CPT * CH
E_PAD = NT * EPT
NPAD = 10240
STRIPE = NPAD // NS
HW = 16

_mesh = plsc.VectorSubcoreMesh(
    core_axis_name="c", subcore_axis_name="s", num_cores=NC, num_subcores=NS
)


@functools.partial(
    pl.kernel,
    out_type=jax.ShapeDtypeStruct((NC, NPAD), jnp.float32),
    mesh=_mesh,
    scratch_types=[
        pltpu.VMEM((EPT,), jnp.int32),
        pltpu.VMEM((NPAD,), jnp.float32),
        pltpu.VMEM((STRIPE,), jnp.float32),
        pltpu.VMEM((STRIPE,), jnp.float32),
        pltpu.VMEM_SHARED((NS, NPAD), jnp.float32),
    ],
    compiler_params=pltpu.CompilerParams(needs_layout_passes=False),
)
def _degree_kernel(row_hbm, out_hbm, ridx_v, hist_v, tmp_v, acc_v, hist_sh):
    c = lax.axis_index("c")
    s = lax.axis_index("s")
    g = c * NS + s
    pltpu.sync_copy(row_hbm.at[g], ridx_v)
    zero = jnp.zeros((16,), jnp.float32)

    def zbody(i, carry):
        hist_v[pl.ds(i * 16, 16)] = zero
        return carry

    lax.fori_loop(0, NPAD // 16, zbody, 0)

    def body(i, carry):
        for u in range(2):
            idx = ridx_v[pl.ds(i * 32 + u * 16, 16)]
            cnt, last = plsc.scan_count(idx)
            plsc.addupdate_scatter(
                hist_v, [idx], cnt.astype(jnp.float32), mask=last
            )
        return carry

    lax.fori_loop(0, EPT // 32, body, 0)
    pltpu.sync_copy(hist_v, hist_sh.at[s])
    plsc.subcore_barrier()

    def zbody2(i, carry):
        acc_v[pl.ds(i * 16, 16)] = zero
        return carry

    lax.fori_loop(0, STRIPE // 16, zbody2, 0)
    for t in range(NS):
        pltpu.sync_copy(hist_sh.at[t, pl.ds(s * STRIPE, STRIPE)], tmp_v)

        def rbody(i, carry):
            acc_v[pl.ds(i * 16, 16)] = (
                acc_v[pl.ds(i * 16, 16)] + tmp_v[pl.ds(i * 16, 16)]
            )
            return carry

        lax.fori_loop(0, STRIPE // 16, rbody, 0)
    pltpu.sync_copy(acc_v, out_hbm.at[c, pl.ds(s * STRIPE, STRIPE)])


def _prescale_body(h_ref, x_ref, y_ref):
    h = h_ref[...]
    deg = h[0:1] + h[1:2]
    safe = jnp.where(deg > 0, deg, 1.0)
    dinv = jnp.where(deg > 0, lax.rsqrt(safe), 0.0)
    dcol = jnp.transpose(dinv)
    y_ref[...] = x_ref[...] * dcol[:N]


@functools.partial(
    pl.kernel,
    out_type=jax.ShapeDtypeStruct((NC, NPAD, D), jnp.float32),
    mesh=_mesh,
    scratch_types=[
        pltpu.VMEM((CPT // 2, CH), jnp.int32),
        pltpu.VMEM((CPT // 2, CH), jnp.int32),
        pltpu.VMEM((CH, D), jnp.float32),
        pltpu.VMEM((CH, D), jnp.float32),
        pltpu.VMEM_SHARED((NPAD, D), jnp.float32),
        pltpu.SemaphoreType.DMA,
        pltpu.SemaphoreType.DMA,
        pltpu.SemaphoreType.DMA,
        pltpu.SemaphoreType.DMA,
    ],
)
def _spmm_kernel(row_hbm, col_hbm, y_hbm, zeros_hbm, out_hbm,
                 ridx_v, cidx_v, gbuf0, gbuf1, acc_sh, sem0, sem1, ssem0, ssem1):
    c = lax.axis_index("c")
    s = lax.axis_index("s")
    g = c * NS + s
    hcpt = CPT // 2
    pltpu.sync_copy(
        zeros_hbm.at[pl.ds(s * STRIPE, STRIPE)],
        acc_sh.at[pl.ds(s * STRIPE, STRIPE)],
    )
    plsc.subcore_barrier()

    for ph in range(2):
        pltpu.sync_copy(row_hbm.at[g, pl.ds(ph * hcpt, hcpt)], ridx_v)
        pltpu.sync_copy(col_hbm.at[g, pl.ds(ph * hcpt, hcpt)], cidx_v)
        pltpu.async_copy(y_hbm.at[cidx_v.at[0]], gbuf0, sem0)
        pltpu.async_copy(y_hbm.at[cidx_v.at[1]], gbuf1, sem1)

        def body(i, carry):
            j0 = 2 * i
            j1 = j0 + 1
            pltpu.make_async_copy(y_hbm.at[cidx_v.at[j0]], gbuf0, sem0).wait()
            d0 = pltpu.async_copy(gbuf0, acc_sh.at[ridx_v.at[j0]], ssem0, add=True)
            pltpu.make_async_copy(y_hbm.at[cidx_v.at[j1]], gbuf1, sem1).wait()
            d1 = pltpu.async_copy(gbuf1, acc_sh.at[ridx_v.at[j1]], ssem1, add=True)
            d0.wait()

            @pl.when(j0 + 2 < hcpt)
            def _():
                pltpu.async_copy(y_hbm.at[cidx_v.at[j0 + 2]], gbuf0, sem0)

            d1.wait()

            @pl.when(j1 + 2 < hcpt)
            def _():
                pltpu.async_copy(y_hbm.at[cidx_v.at[j1 + 2]], gbuf1, sem1)

            return carry

        lax.fori_loop(0, hcpt // 2, body, 0)
    plsc.subcore_barrier()
    pltpu.sync_copy(
        acc_sh.at[pl.ds(s * STRIPE, STRIPE)],
        out_hbm.at[c, pl.ds(s * STRIPE, STRIPE)],
    )


def _post_body(h_ref, p_ref, o_ref):
    h = h_ref[...]
    deg = h[0:1] + h[1:2]
    safe = jnp.where(deg > 0, deg, 1.0)
    dinv = jnp.where(deg > 0, lax.rsqrt(safe), 0.0)
    dcol = jnp.transpose(dinv)
    o_ref[...] = (p_ref[0, :N, :] + p_ref[1, :N, :]) * dcol[:N]


def kernel(features, edge_index):
    features = features.astype(jnp.float32)
    row = edge_index[0].astype(jnp.int32)
    col = edge_index[1].astype(jnp.int32)

    npad = E_PAD - E
    pidx = jnp.arange(npad, dtype=jnp.int32)
    prow = N + (pidx % (NPAD - N))
    pcol = (pidx * 37) % N
    rowp = jnp.concatenate([row, prow]).reshape(NT, CPT, CH)
    colp = jnp.concatenate([col, pcol]).reshape(NT, CPT, CH)

    zeros = jnp.zeros((NPAD, D), jnp.float32)

    hist = _degree_kernel(rowp.reshape(NT, EPT))

    y = pl.pallas_call(
        _prescale_body,
        out_shape=jax.ShapeDtypeStruct((N, D), jnp.float32),
    )(hist, features)

    partials = _spmm_kernel(rowp, colp, y, zeros)

    out = pl.pallas_call(
        _post_body,
        out_shape=jax.ShapeDtypeStruct((N, D), jnp.float32),
    )(hist, partials)
    return out


# speedup vs baseline: 1.2011x; 1.2011x over previous
import functools

import jax
import jax.numpy as jnp
from jax import lax
from jax.experimental import pallas as pl
from jax.experimental.pallas import tpu as pltpu
from jax.experimental.pallas import tpu_sc as plsc

N = 10000
E = 320000
D = 128
NC = 2
NS = 16
NT = NC * NS
CH = 128
CPT = 80
EPT = CPT * CH
E_PAD = NT * EPT
NPAD = 10240
STRIPE = NPAD // NS
HW = 16

_mesh = plsc.VectorSubcoreMesh(
    core_axis_name="c", subcore_axis_name="s", num_cores=NC, num_subcores=NS
)


@functools.partial(
    pl.kernel,
    out_type=jax.ShapeDtypeStruct((NC, NPAD), jnp.float32),
    mesh=_mesh,
    scratch_types=[
        pltpu.VMEM((EPT,), jnp.int32),
        pltpu.VMEM((NPAD,), jnp.float32),
        pltpu.VMEM((STRIPE,), jnp.float32),
        pltpu.VMEM((STRIPE,), jnp.float32),
        pltpu.VMEM_SHARED((NS, NPAD), jnp.float32),
    ],
    compiler_params=pltpu.CompilerParams(needs_layout_passes=False),
)
def _degree_kernel(row_hbm, out_hbm, ridx_v, hist_v, tmp_v, acc_v, hist_sh):
    c = lax.axis_index("c")
    s = lax.axis_index("s")
    g = c * NS + s
    pltpu.sync_copy(row_hbm.at[g], ridx_v)
    zero = jnp.zeros((16,), jnp.float32)

    def zbody(i, carry):
        hist_v[pl.ds(i * 16, 16)] = zero
        return carry

    lax.fori_loop(0, NPAD // 16, zbody, 0)

    def body(i, carry):
        for u in range(2):
            idx = ridx_v[pl.ds(i * 32 + u * 16, 16)]
            cnt, last = plsc.scan_count(idx)
            plsc.addupdate_scatter(
                hist_v, [idx], cnt.astype(jnp.float32), mask=last
            )
        return carry

    lax.fori_loop(0, EPT // 32, body, 0)
    pltpu.sync_copy(hist_v, hist_sh.at[s])
    plsc.subcore_barrier()

    def zbody2(i, carry):
        acc_v[pl.ds(i * 16, 16)] = zero
        return carry

    lax.fori_loop(0, STRIPE // 16, zbody2, 0)
    for t in range(NS):
        pltpu.sync_copy(hist_sh.at[t, pl.ds(s * STRIPE, STRIPE)], tmp_v)

        def rbody(i, carry):
            acc_v[pl.ds(i * 16, 16)] = (
                acc_v[pl.ds(i * 16, 16)] + tmp_v[pl.ds(i * 16, 16)]
            )
            return carry

        lax.fori_loop(0, STRIPE // 16, rbody, 0)
    pltpu.sync_copy(acc_v, out_hbm.at[c, pl.ds(s * STRIPE, STRIPE)])


def _prescale_body(h_ref, x_ref, y_ref):
    h = h_ref[...]
    deg = h[0:1] + h[1:2]
    safe = jnp.where(deg > 0, deg, 1.0)
    dinv = jnp.where(deg > 0, lax.rsqrt(safe), 0.0)
    dcol = jnp.transpose(dinv)
    y_ref[...] = x_ref[...] * dcol[:N]


@functools.partial(
    pl.kernel,
    out_type=jax.ShapeDtypeStruct((NC, NPAD, D), jnp.float32),
    mesh=_mesh,
    scratch_types=[
        pltpu.VMEM((CPT // 2, CH), jnp.int32),
        pltpu.VMEM((CPT // 2, CH), jnp.int32),
        pltpu.VMEM((CH, D), jnp.float32),
        pltpu.VMEM((CH, D), jnp.float32),
        pltpu.VMEM_SHARED((NPAD, D), jnp.float32),
        pltpu.SemaphoreType.DMA,
        pltpu.SemaphoreType.DMA,
        pltpu.SemaphoreType.DMA,
        pltpu.SemaphoreType.DMA,
    ],
)
def _spmm_kernel(row_hbm, col_hbm, y_hbm, zeros_hbm, out_hbm,
                 ridx_v, cidx_v, gbuf0, gbuf1, acc_sh, sem0, sem1, ssem0, ssem1):
    c = lax.axis_index("c")
    s = lax.axis_index("s")
    g = c * NS + s
    hcpt = CPT // 2
    pltpu.sync_copy(
        zeros_hbm.at[pl.ds(s * STRIPE, STRIPE)],
        acc_sh.at[pl.ds(s * STRIPE, STRIPE)],
    )
    plsc.subcore_barrier()

    for ph in range(2):
        pltpu.sync_copy(row_hbm.at[g, pl.ds(ph * hcpt, hcpt)], ridx_v)
        pltpu.sync_copy(col_hbm.at[g, pl.ds(ph * hcpt, hcpt)], cidx_v)
        pltpu.async_copy(y_hbm.at[cidx_v.at[0]], gbuf0, sem0)

        def body(i, carry):
            j0 = 2 * i
            j1 = j0 + 1
            pltpu.async_copy(y_hbm.at[cidx_v.at[j1]], gbuf1, sem1)
            pltpu.make_async_copy(y_hbm.at[cidx_v.at[j0]], gbuf0, sem0).wait()
            pltpu.sync_copy(gbuf0, acc_sh.at[ridx_v.at[j0]], add=True)

            @pl.when(j0 + 2 < hcpt)
            def _():
                pltpu.async_copy(y_hbm.at[cidx_v.at[j0 + 2]], gbuf0, sem0)

            pltpu.make_async_copy(y_hbm.at[cidx_v.at[j1]], gbuf1, sem1).wait()
            pltpu.sync_copy(gbuf1, acc_sh.at[ridx_v.at[j1]], add=True)
            return carry

        lax.fori_loop(0, hcpt // 2, body, 0)
    plsc.subcore_barrier()
    pltpu.sync_copy(
        acc_sh.at[pl.ds(s * STRIPE, STRIPE)],
        out_hbm.at[c, pl.ds(s * STRIPE, STRIPE)],
    )


def _post_body(h_ref, p_ref, o_ref):
    h = h_ref[...]
    deg = h[0:1] + h[1:2]
    safe = jnp.where(deg > 0, deg, 1.0)
    dinv = jnp.where(deg > 0, lax.rsqrt(safe), 0.0)
    dcol = jnp.transpose(dinv)
    o_ref[...] = (p_ref[0, :N, :] + p_ref[1, :N, :]) * dcol[:N]


def kernel(features, edge_index):
    features = features.astype(jnp.float32)
    row = edge_index[0].astype(jnp.int32)
    col = edge_index[1].astype(jnp.int32)

    npad = E_PAD - E
    pidx = jnp.arange(npad, dtype=jnp.int32)
    prow = N + (pidx % (NPAD - N))
    pcol = (pidx * 37) % N
    rowp = jnp.concatenate([row, prow]).reshape(NT, CPT, CH)
    colp = jnp.concatenate([col, pcol]).reshape(NT, CPT, CH)

    zeros = jnp.zeros((NPAD, D), jnp.float32)

    hist = _degree_kernel(rowp.reshape(NT, EPT))

    y = pl.pallas_call(
        _prescale_body,
        out_shape=jax.ShapeDtypeStruct((N, D), jnp.float32),
    )(hist, features)

    partials = _spmm_kernel(rowp, colp, y, zeros)

    out = pl.pallas_call(
        _post_body,
        out_shape=jax.ShapeDtypeStruct((N, D), jnp.float32),
    )(hist, partials)
    return out


# speedup vs baseline: 1.2369x; 1.0298x over previous
import functools

import jax
import jax.numpy as jnp
import numpy as np
from jax import lax
from jax.experimental import pallas as pl
from jax.experimental.pallas import tpu as pltpu
from jax.experimental.pallas import tpu_sc as plsc

N = 10000
E = 320000
D = 128
NC = 2
NS = 16
NT = NC * NS
CH = 128
CPT = 80
EPT = CPT * CH
ECH = E // CH
RCH31 = ECH - (NT - 1) * CPT
PCH = NT * CPT - ECH
NPAD = 10240
STRIPE = NPAD // NS

_PROW = jnp.asarray(
    (N + np.arange(PCH * CH) % (NPAD - N)).reshape(PCH, CH).astype(np.int32)
)
_PCOL = jnp.asarray(
    ((np.arange(PCH * CH) * 37) % N).reshape(PCH, CH).astype(np.int32)
)

_mesh = plsc.VectorSubcoreMesh(
    core_axis_name="c", subcore_axis_name="s", num_cores=NC, num_subcores=NS
)


@functools.partial(
    pl.kernel,
    out_type=jax.ShapeDtypeStruct((NC, NPAD), jnp.float32),
    mesh=_mesh,
    scratch_types=[
        pltpu.VMEM((CPT, CH), jnp.int32),
        pltpu.VMEM((NPAD,), jnp.float32),
        pltpu.VMEM((STRIPE,), jnp.float32),
        pltpu.VMEM((STRIPE,), jnp.float32),
        pltpu.VMEM_SHARED((NS, NPAD), jnp.float32),
    ],
    compiler_params=pltpu.CompilerParams(needs_layout_passes=False),
)
def _degree_kernel(row_hbm, out_hbm, ridx_v, hist_v, tmp_v, acc_v, hist_sh):
    c = lax.axis_index("c")
    s = lax.axis_index("s")
    g = c * NS + s
    pltpu.sync_copy(row_hbm.at[pl.ds(g * CPT, CPT)], ridx_v)
    zero = jnp.zeros((16,), jnp.float32)

    def zbody(i, carry):
        for u in range(8):
            hist_v[pl.ds((i * 8 + u) * 16, 16)] = zero
        return carry

    lax.fori_loop(0, NPAD // 128, zbody, 0)

    def body(j, carry):
        for u in range(CH // 16):
            idx = ridx_v[j, pl.ds(u * 16, 16)]
            cnt, last = plsc.scan_count(idx)
            plsc.addupdate_scatter(
                hist_v, [idx], cnt.astype(jnp.float32), mask=last
            )
        return carry

    lax.fori_loop(0, CPT, body, 0)
    pltpu.sync_copy(hist_v, hist_sh.at[s])
    plsc.subcore_barrier()

    def zbody2(i, carry):
        for u in range(4):
            acc_v[pl.ds((i * 4 + u) * 16, 16)] = zero
        return carry

    lax.fori_loop(0, STRIPE // 64, zbody2, 0)
    for t in range(NS):
        pltpu.sync_copy(hist_sh.at[t, pl.ds(s * STRIPE, STRIPE)], tmp_v)

        def rbody(i, carry):
            for u in range(4):
                off = (i * 4 + u) * 16
                acc_v[pl.ds(off, 16)] = acc_v[pl.ds(off, 16)] + tmp_v[pl.ds(off, 16)]
            return carry

        lax.fori_loop(0, STRIPE // 64, rbody, 0)
    pltpu.sync_copy(acc_v, out_hbm.at[c, pl.ds(s * STRIPE, STRIPE)])


def _prescale_body(h_ref, x_ref, y_ref):
    h = h_ref[...]
    deg = h[0:1] + h[1:2]
    safe = jnp.where(deg > 0, deg, 1.0)
    dinv = jnp.where(deg > 0, lax.rsqrt(safe), 0.0)
    dcol = jnp.transpose(dinv)
    y_ref[...] = x_ref[...] * dcol[:N]


@functools.partial(
    pl.kernel,
    out_type=jax.ShapeDtypeStruct((NC, NPAD, D), jnp.float32),
    mesh=_mesh,
    scratch_types=[
        pltpu.VMEM((CPT // 2, CH), jnp.int32),
        pltpu.VMEM((CPT // 2, CH), jnp.int32),
        pltpu.VMEM((CH, D), jnp.float32),
        pltpu.VMEM((CH, D), jnp.float32),
        pltpu.VMEM_SHARED((NPAD, D), jnp.float32),
        pltpu.SemaphoreType.DMA,
        pltpu.SemaphoreType.DMA,
    ],
)
def _spmm_kernel(row_hbm, col_hbm, y_hbm, zeros_hbm,
                 out_hbm, ridx_v, cidx_v, gbuf0, gbuf1, acc_sh, sem0, sem1):
    c = lax.axis_index("c")
    s = lax.axis_index("s")
    g = c * NS + s
    hcpt = CPT // 2
    pltpu.sync_copy(row_hbm.at[pl.ds(g * CPT, hcpt)], ridx_v)
    pltpu.sync_copy(col_hbm.at[pl.ds(g * CPT, hcpt)], cidx_v)
    pltpu.async_copy(y_hbm.at[cidx_v.at[0]], gbuf0, sem0)
    pltpu.sync_copy(
        zeros_hbm.at[pl.ds(s * STRIPE, STRIPE)],
        acc_sh.at[pl.ds(s * STRIPE, STRIPE)],
    )
    plsc.subcore_barrier()

    for ph in range(2):
        if ph:
            pltpu.sync_copy(row_hbm.at[pl.ds(g * CPT + ph * hcpt, hcpt)], ridx_v)
            pltpu.sync_copy(col_hbm.at[pl.ds(g * CPT + ph * hcpt, hcpt)], cidx_v)
            pltpu.async_copy(y_hbm.at[cidx_v.at[0]], gbuf0, sem0)

        def body(i, carry):
            j0 = 2 * i
            j1 = j0 + 1
            pltpu.async_copy(y_hbm.at[cidx_v.at[j1]], gbuf1, sem1)
            pltpu.make_async_copy(y_hbm.at[cidx_v.at[j0]], gbuf0, sem0).wait()
            pltpu.sync_copy(gbuf0, acc_sh.at[ridx_v.at[j0]], add=True)

            @pl.when(j0 + 2 < hcpt)
            def _():
                pltpu.async_copy(y_hbm.at[cidx_v.at[j0 + 2]], gbuf0, sem0)

            pltpu.make_async_copy(y_hbm.at[cidx_v.at[j1]], gbuf1, sem1).wait()
            pltpu.sync_copy(gbuf1, acc_sh.at[ridx_v.at[j1]], add=True)
            return carry

        lax.fori_loop(0, hcpt // 2, body, 0)
    plsc.subcore_barrier()
    pltpu.sync_copy(
        acc_sh.at[pl.ds(s * STRIPE, STRIPE)],
        out_hbm.at[c, pl.ds(s * STRIPE, STRIPE)],
    )


def _post_body(h_ref, p_ref, o_ref):
    h = h_ref[...]
    deg = h[0:1] + h[1:2]
    safe = jnp.where(deg > 0, deg, 1.0)
    dinv = jnp.where(deg > 0, lax.rsqrt(safe), 0.0)
    dcol = jnp.transpose(dinv)
    o_ref[...] = (p_ref[0, :N, :] + p_ref[1, :N, :]) * dcol[:N]


def kernel(features, edge_index):
    features = features.astype(jnp.float32)
    row2 = edge_index[0].astype(jnp.int32).reshape(ECH, CH)
    col2 = edge_index[1].astype(jnp.int32).reshape(ECH, CH)

    zeros = jnp.zeros((NPAD, D), jnp.float32)
    rowp = jnp.concatenate([row2, _PROW], axis=0)
    colp = jnp.concatenate([col2, _PCOL], axis=0)

    hist = _degree_kernel(rowp)

    y = pl.pallas_call(
        _prescale_body,
        out_shape=jax.ShapeDtypeStruct((N, D), jnp.float32),
    )(hist, features)

    partials = _spmm_kernel(rowp, colp, y, zeros)

    out = pl.pallas_call(
        _post_body,
        out_shape=jax.ShapeDtypeStruct((N, D), jnp.float32),
    )(hist, partials)
    return out
